# baseline (speedup 1.0000x reference)
import functools

import jax
import jax.numpy as jnp
from jax import lax
from jax.experimental import pallas as pl
from jax.experimental.pallas import tpu as pltpu
from jax.experimental.pallas import tpu_sc as plsc

B, C, K = 4096, 26, 1000
NC, NS = 2, 16
NW = NC * NS
BPW = B // NW
L = 16
KT = K // 8
G = 25
NG = KT // G
NU = C * NG

_mesh = plsc.VectorSubcoreMesh(core_axis_name="c", subcore_axis_name="s")


@functools.partial(
    pl.kernel,
    mesh=_mesh,
    out_type=jax.ShapeDtypeStruct((C, K, B), jnp.int32),
    compiler_params=pltpu.CompilerParams(
        needs_layout_passes=False, disable_bounds_checks=True),
    scratch_types=[
        pltpu.VMEM((C, BPW), jnp.int32),
        pltpu.VMEM((G * 8, 128), jnp.int32),
        pltpu.VMEM((G * 8, 128), jnp.int32),
        pltpu.SemaphoreType.DMA,
        pltpu.SemaphoreType.DMA,
    ],
)
def _onehot_sc(x_hbm, out_hbm, xl, buf_a, buf_b, sem_a, sem_b):
    wid = lax.axis_index("s") * NC + lax.axis_index("c")
    b0 = wid * BPW
    bs0 = pl.multiple_of(b0, 128)

    pltpu.sync_copy(x_hbm.at[pl.ds(0, 8), pl.ds(bs0, BPW)],
                    xl.at[pl.ds(0, 8)])
    pltpu.sync_copy(x_hbm.at[pl.ds(8, 8), pl.ds(bs0, BPW)],
                    xl.at[pl.ds(8, 8)])
    pltpu.sync_copy(x_hbm.at[pl.ds(16, 8), pl.ds(bs0, BPW)],
                    xl.at[pl.ds(16, 8)])
    pltpu.sync_copy(x_hbm.at[pl.ds(24, 2), pl.ds(bs0, BPW)],
                    xl.at[pl.ds(24, 2)])

    zeros = jnp.zeros((L,), jnp.int32)
    ones = jnp.ones((L,), jnp.int32)
    iota = lax.iota(jnp.int32, L)

    def zfill(r, _):
        def zfill_chunk(j, _):
            o = pl.multiple_of(j * L, L)
            buf_a[r, pl.ds(o, L)] = zeros
            buf_b[r, pl.ds(o, L)] = zeros
            return 0
        return lax.fori_loop(0, 128 // L, zfill_chunk, 0)

    lax.fori_loop(0, G * 8, zfill, 0)

    def scatter(buf, u, what):
        c = u // NG
        kt0 = (u % NG) * G

        cvec = jnp.full((L,), c, jnp.int32)

        def chunk(j, _):
            lanes = j * L + iota
            v = plsc.load_gather(xl, [cvec, lanes])
            kt = v >> 3
            m = (kt >= kt0) & (kt < kt0 + G)
            plsc.store_scatter(buf, [(kt - kt0) * 8 + (v & 7), lanes], what,
                               mask=m)
            return 0

        lax.fori_loop(0, BPW // L, chunk, 0)

    def fire(buf, u, sem):
        scatter(buf, u, ones)
        c = u // NG
        kt0 = (u % NG) * G

        def issue(t, _):
            ks = pl.multiple_of((kt0 + t) * 8, 8)
            bs = pl.multiple_of(b0, 128)
            pltpu.async_copy(
                buf.at[pl.ds(pl.multiple_of(t * 8, 8), 8)],
                out_hbm.at[c, pl.ds(ks, 8), pl.ds(bs, 128)], sem)
            return 0

        lax.fori_loop(0, G, issue, 0)

    def drain(buf, sem):
        pltpu.make_async_copy(
            out_hbm.at[0, pl.ds(0, G * 8), pl.ds(0, 128)], buf, sem).wait()

    fire(buf_a, 0, sem_a)
    fire(buf_b, 1, sem_b)

    def step(p, _):
        u = 2 * p
        drain(buf_a, sem_a)
        scatter(buf_a, u - 2, zeros)
        fire(buf_a, u, sem_a)
        drain(buf_b, sem_b)
        scatter(buf_b, u - 1, zeros)
        fire(buf_b, u + 1, sem_b)
        return 0

    lax.fori_loop(1, NU // 2, step, 0)

    drain(buf_a, sem_a)
    drain(buf_b, sem_b)


def kernel(x):
    out = _onehot_sc(x.T)
    return jnp.transpose(out, (2, 0, 1))


# speedup vs baseline: 1.0442x; 1.0442x over previous
import functools

import jax
import jax.numpy as jnp
from jax import lax
from jax.experimental import pallas as pl
from jax.experimental.pallas import tpu as pltpu
from jax.experimental.pallas import tpu_sc as plsc

B, C, K = 4096, 26, 1000
CP = 32
NC, NS = 2, 16
NW = NC * NS
BPW = B // NW
L = 16
KT = K // 8
G = 25
NG = KT // G
NU = C * NG

_mesh = plsc.VectorSubcoreMesh(core_axis_name="c", subcore_axis_name="s")


@functools.partial(
    pl.kernel,
    mesh=_mesh,
    out_type=jax.ShapeDtypeStruct((C, K, B), jnp.int32),
    compiler_params=pltpu.CompilerParams(
        needs_layout_passes=False, disable_bounds_checks=True),
    scratch_types=[
        pltpu.VMEM((BPW * CP,), jnp.int32),
        pltpu.VMEM((G * 8, 128), jnp.int32),
        pltpu.VMEM((G * 8, 128), jnp.int32),
        pltpu.SemaphoreType.DMA,
        pltpu.SemaphoreType.DMA,
    ],
)
def _onehot_sc(x_hbm, out_hbm, xl, buf_a, buf_b, sem_a, sem_b):
    wid = lax.axis_index("s") * NC + lax.axis_index("c")
    b0 = wid * BPW

    pltpu.sync_copy(x_hbm.at[pl.ds(b0 * CP, BPW * CP)], xl)

    zeros = jnp.zeros((L,), jnp.int32)
    ones = jnp.ones((L,), jnp.int32)
    iota = lax.iota(jnp.int32, L)

    def zfill(r, _):
        def zfill_chunk(j, _):
            o = pl.multiple_of(j * L, L)
            buf_a[r, pl.ds(o, L)] = zeros
            buf_b[r, pl.ds(o, L)] = zeros
            return 0
        return lax.fori_loop(0, 128 // L, zfill_chunk, 0)

    lax.fori_loop(0, G * 8, zfill, 0)

    def scatter(buf, u, what):
        c = u // NG
        kt0 = (u % NG) * G

        def chunk(j, _):
            lanes = j * L + iota
            v = plsc.load_gather(xl, [lanes * CP + c])
            kt = v >> 3
            m = (kt >= kt0) & (kt < kt0 + G)
            plsc.store_scatter(buf, [(kt - kt0) * 8 + (v & 7), lanes], what,
                               mask=m)
            return 0

        lax.fori_loop(0, BPW // L, chunk, 0)

    def fire(buf, u, sem):
        scatter(buf, u, ones)
        c = u // NG
        kt0 = (u % NG) * G

        def issue(t, _):
            ks = pl.multiple_of((kt0 + t) * 8, 8)
            bs = pl.multiple_of(b0, 128)
            pltpu.async_copy(
                buf.at[pl.ds(pl.multiple_of(t * 8, 8), 8)],
                out_hbm.at[c, pl.ds(ks, 8), pl.ds(bs, 128)], sem)
            return 0

        lax.fori_loop(0, G, issue, 0)

    def drain(buf, sem):
        pltpu.make_async_copy(
            out_hbm.at[0, pl.ds(0, G * 8), pl.ds(0, 128)], buf, sem).wait()

    fire(buf_a, 0, sem_a)
    fire(buf_b, 1, sem_b)

    def step(p, _):
        u = 2 * p
        drain(buf_a, sem_a)
        scatter(buf_a, u - 2, zeros)
        fire(buf_a, u, sem_a)
        drain(buf_b, sem_b)
        scatter(buf_b, u - 1, zeros)
        fire(buf_b, u + 1, sem_b)
        return 0

    lax.fori_loop(1, NU // 2, step, 0)

    drain(buf_a, sem_a)
    drain(buf_b, sem_b)


def kernel(x):
    xp = jnp.pad(x, ((0, 0), (0, CP - C)))
    out = _onehot_sc(xp.reshape(B * CP))
    return jnp.transpose(out, (2, 0, 1))
